# Initial kernel scaffold; baseline (speedup 1.0000x reference)
import jax
import jax.numpy as jnp
from jax.experimental import pallas as pl


def kernel(x, batch, W1, b1, g1, be1, W2, b2, g2, be2, W3, b3, g3, be3, W4, b4, g4, be4, Wl, bl):
    raise NotImplementedError("write your pallas kernel here")


# speedup vs baseline: 8.0833x; 8.0833x over previous
import functools

import jax
import jax.numpy as jnp
from jax import lax
from jax.experimental import pallas as pl
from jax.experimental.pallas import tpu as pltpu
from jax.experimental.pallas import tpu_sc as plsc

N = 10000
NP = 10240
K = 20
G = 10
M = N * K
MF = float(M)
T = 512
NT = NP // T
TE = 2560
NTE = NP // TE
_INF = float("inf")
_IMAX = 2**31 - 1
_EPS = 1e-5


def _knn_body(rmin_ref, rmax_ref, x_ref, xT_ref, bcol_ref, brow_ref,
              out_ref, topd, topi):
    r = pl.program_id(0)
    c = pl.program_id(1)

    @pl.when(c == 0)
    def _init():
        topd[...] = jnp.full((T, 32), _INF, jnp.float32)
        topi[...] = jnp.full((T, 32), _IMAX, jnp.int32)

    active = jnp.logical_and(rmin_ref[c] <= rmax_ref[r],
                             rmax_ref[c] >= rmin_ref[r])

    @pl.when(active)
    def _merge():
        xr = x_ref[pl.ds(r * T, T), :]
        xc = xT_ref[:, pl.ds(c * T, T)]
        xxr = jnp.sum(xr * xr, axis=1, keepdims=True)
        xxc = jnp.sum(xc * xc, axis=0, keepdims=True)
        d = xxr - 2.0 * jnp.dot(xr, xc, preferred_element_type=jnp.float32) + xxc
        same = bcol_ref[pl.ds(r * T, T), :] == brow_ref[:, pl.ds(c * T, T)]
        d = jnp.where(same, d, _INF)
        ids = c * T + lax.broadcasted_iota(jnp.int32, (T, T), 1)
        cd = jnp.concatenate([topd[...], d], axis=1)
        ci = jnp.concatenate([topi[...], ids], axis=1)
        nd = []
        ni = []
        for _ in range(K):
            m = jnp.min(cd, axis=1, keepdims=True)
            sel = jnp.min(jnp.where(cd == m, ci, _IMAX), axis=1, keepdims=True)
            nd.append(m)
            ni.append(sel)
            cd = jnp.where(ci == sel, _INF, cd)
        pad_d = jnp.full((T, 32 - K), _INF, jnp.float32)
        pad_i = jnp.full((T, 32 - K), _IMAX, jnp.int32)
        topd[...] = jnp.concatenate(nd + [pad_d], axis=1)
        topi[...] = jnp.concatenate(ni + [pad_i], axis=1)

    out_ref[...] = topi[...]


def _knn(xp, xpT, bcol, brow, rmin, rmax):
    return pl.pallas_call(
        _knn_body,
        grid=(NT, NT),
        in_specs=[
            pl.BlockSpec(memory_space=pltpu.SMEM),
            pl.BlockSpec(memory_space=pltpu.SMEM),
            pl.BlockSpec((NP, 128), lambda r, c: (0, 0)),
            pl.BlockSpec((128, NP), lambda r, c: (0, 0)),
            pl.BlockSpec((NP, 1), lambda r, c: (0, 0)),
            pl.BlockSpec((1, NP), lambda r, c: (0, 0)),
        ],
        out_specs=pl.BlockSpec((T, 32), lambda r, c: (r, 0)),
        out_shape=jax.ShapeDtypeStruct((NP, 32), jnp.int32),
        scratch_shapes=[
            pltpu.VMEM((T, 32), jnp.float32),
            pltpu.VMEM((T, 32), jnp.int32),
        ],
    )(rmin, rmax, xp, xpT, bcol, brow)


def _sc_gather(table, idx, chunk):
    b = idx.shape[0]
    d = table.shape[1]
    nw = 32
    b_per_w = b // nw
    nchunks = b_per_w // chunk
    mesh = plsc.VectorSubcoreMesh(core_axis_name="c", subcore_axis_name="s")

    @functools.partial(
        pl.kernel,
        mesh=mesh,
        out_type=jax.ShapeDtypeStruct((b, d), jnp.float32),
        scratch_types=[
            pltpu.VMEM((chunk,), jnp.int32),
            pltpu.VMEM((chunk, d), jnp.float32),
            pltpu.SemaphoreType.DMA,
        ],
    )
    def gather_kernel(table_hbm, idx_hbm, out_hbm, idx_v, rows_v, sem):
        wid = lax.axis_index("s") * 2 + lax.axis_index("c")
        base = wid * b_per_w
        for ci in range(nchunks):
            off = base + ci * chunk
            pltpu.sync_copy(idx_hbm.at[pl.ds(off, chunk)], idx_v)
            pltpu.async_copy(table_hbm.at[idx_v], rows_v, sem).wait()
            pltpu.sync_copy(rows_v, out_hbm.at[pl.ds(off, chunk)])

    return gather_kernel(table, idx)


def _mlp1_body(x_ref, xj_ref, w1_ref, w2_ref, w3_ref, pv_ref,
               out_ref, stats):
    p = pl.program_id(0)
    k = pl.program_id(1)
    t = pl.program_id(2)

    @pl.when(jnp.logical_and(p == 0, jnp.logical_and(k == 0, t == 0)))
    def _init():
        stats[...] = jnp.zeros((8, 64), jnp.float32)

    x_t = x_ref[pl.ds(t * TE, TE), :]
    xj = xj_ref[...]
    col = lax.broadcasted_iota(jnp.int32, (1, 128), 1)
    e = jnp.where(col < 8, x_t, 0.0) + jnp.where(
        jnp.logical_and(col >= 8, col < 16), xj - x_t, 0.0)
    a1 = jnp.maximum(
        jnp.dot(e, w1_ref[...], preferred_element_type=jnp.float32)
        + pv_ref[0:1, :], 0.0)
    rows = t * TE + lax.broadcasted_iota(jnp.int32, (TE, 1), 0)
    mask = rows < N

    def msum(v):
        return jnp.sum(jnp.where(mask, v, 0.0), axis=0, keepdims=True)

    def mean_of(srow):
        return stats[srow:srow + 1, :] / MF

    def bn(a, srow, grow, berow):
        m = mean_of(srow)
        v = stats[srow + 1:srow + 2, :] / MF
        return (pv_ref[grow:grow + 1, :] * (a - m) * lax.rsqrt(v + _EPS)
                + pv_ref[berow:berow + 1, :])

    @pl.when(p == 0)
    def _sum1():
        stats[0:1, :] += msum(a1)

    @pl.when(p == 1)
    def _var1():
        d1 = a1 - mean_of(0)
        stats[1:2, :] += msum(d1 * d1)

    @pl.when(p >= 2)
    def _l2():
        h1n = bn(a1, 0, 1, 2)
        a2 = jnp.maximum(
            jnp.dot(h1n, w2_ref[...], preferred_element_type=jnp.float32)
            + pv_ref[3:4, :], 0.0)

        @pl.when(p == 2)
        def _sum2():
            stats[2:3, :] += msum(a2)

        @pl.when(p == 3)
        def _var2():
            d2 = a2 - mean_of(2)
            stats[3:4, :] += msum(d2 * d2)

        @pl.when(p >= 4)
        def _l3():
            h2n = bn(a2, 2, 4, 5)
            a3 = jnp.maximum(
                jnp.dot(h2n, w3_ref[...], preferred_element_type=jnp.float32)
                + pv_ref[6:7, :], 0.0)

            @pl.when(p == 4)
            def _sum3():
                stats[4:5, :] += msum(a3)

            @pl.when(p == 5)
            def _var3():
                d3 = a3 - mean_of(4)
                stats[5:6, :] += msum(d3 * d3)

            @pl.when(p == 6)
            def _final():
                h3 = bn(a3, 4, 7, 8)
                h3p = jnp.concatenate([h3, h3], axis=1)

                @pl.when(k == 0)
                def _first():
                    out_ref[pl.ds(t * TE, TE), :] = h3p

                @pl.when(k > 0)
                def _acc():
                    cur = out_ref[pl.ds(t * TE, TE), :]
                    out_ref[pl.ds(t * TE, TE), :] = jnp.maximum(cur, h3p)


def _mlp1(xp2, xj, w1, w2, w3, pv):
    return pl.pallas_call(
        _mlp1_body,
        grid=(7, K, NTE),
        in_specs=[
            pl.BlockSpec((NP, 128), lambda p, k, t: (0, 0)),
            pl.BlockSpec((TE, 128), lambda p, k, t: (k * NTE + t, 0)),
            pl.BlockSpec((128, 64), lambda p, k, t: (0, 0)),
            pl.BlockSpec((64, 64), lambda p, k, t: (0, 0)),
            pl.BlockSpec((64, 64), lambda p, k, t: (0, 0)),
            pl.BlockSpec((16, 64), lambda p, k, t: (0, 0)),
        ],
        out_specs=pl.BlockSpec((NP, 128), lambda p, k, t: (0, 0)),
        out_shape=jax.ShapeDtypeStruct((NP, 128), jnp.float32),
        scratch_shapes=[pltpu.VMEM((8, 64), jnp.float32)],
    )(xp2, xj, w1, w2, w3, pv)


def _mlp2_body(h_ref, hj_ref, w4_ref, pv_ref, wl_ref, bcol_ref,
               out_ref, stats, h2acc, pooled):
    p = pl.program_id(0)
    k = pl.program_id(1)
    t = pl.program_id(2)

    @pl.when(jnp.logical_and(p == 0, jnp.logical_and(k == 0, t == 0)))
    def _init():
        stats[...] = jnp.zeros((8, 128), jnp.float32)

    h_t = h_ref[pl.ds(t * TE, TE), :]
    hj = hj_ref[...]
    col = lax.broadcasted_iota(jnp.int32, (1, 128), 1)
    e = jnp.where(col < 64, h_t, 0.0) + jnp.where(col >= 64, hj - h_t, 0.0)
    a4 = jnp.maximum(
        jnp.dot(e, w4_ref[...], preferred_element_type=jnp.float32)
        + pv_ref[0:1, :], 0.0)
    rows = t * TE + lax.broadcasted_iota(jnp.int32, (TE, 1), 0)
    mask = rows < N

    def msum(v):
        return jnp.sum(jnp.where(mask, v, 0.0), axis=0, keepdims=True)

    @pl.when(p == 0)
    def _sum4():
        stats[0:1, :] += msum(a4)

    @pl.when(p == 1)
    def _var4():
        d4 = a4 - stats[0:1, :] / MF
        stats[1:2, :] += msum(d4 * d4)

    @pl.when(p == 2)
    def _final():
        m = stats[0:1, :] / MF
        v = stats[1:2, :] / MF
        h = pv_ref[1:2, :] * (a4 - m) * lax.rsqrt(v + _EPS) + pv_ref[2:3, :]

        @pl.when(k == 0)
        def _first():
            h2acc[pl.ds(t * TE, TE), :] = h

        @pl.when(k > 0)
        def _acc():
            cur = h2acc[pl.ds(t * TE, TE), :]
            h2acc[pl.ds(t * TE, TE), :] = jnp.maximum(cur, h)

        @pl.when(k == K - 1)
        def _pool():
            @pl.when(t == 0)
            def _pinit():
                pooled[...] = jnp.full((16, 128), -_INF, jnp.float32)

            hcur = h2acc[pl.ds(t * TE, TE), :]
            bt = bcol_ref[pl.ds(t * TE, TE), :]
            for g in range(G):
                sel = jnp.logical_and(bt == g, mask)
                mg = jnp.max(jnp.where(sel, hcur, -_INF), axis=0, keepdims=True)
                pooled[g:g + 1, :] = jnp.maximum(pooled[g:g + 1, :], mg)

            @pl.when(t == NTE - 1)
            def _head():
                out_ref[...] = jnp.dot(
                    pooled[...], wl_ref[...],
                    preferred_element_type=jnp.float32) + pv_ref[3:4, :]


def _mlp2(hp, hj, w4, pv, wl, bcol):
    return pl.pallas_call(
        _mlp2_body,
        grid=(3, K, NTE),
        in_specs=[
            pl.BlockSpec((NP, 128), lambda p, k, t: (0, 0)),
            pl.BlockSpec((TE, 128), lambda p, k, t: (k * NTE + t, 0)),
            pl.BlockSpec((128, 128), lambda p, k, t: (0, 0)),
            pl.BlockSpec((8, 128), lambda p, k, t: (0, 0)),
            pl.BlockSpec((128, 128), lambda p, k, t: (0, 0)),
            pl.BlockSpec((NP, 1), lambda p, k, t: (0, 0)),
        ],
        out_specs=pl.BlockSpec((16, 128), lambda p, k, t: (0, 0)),
        out_shape=jax.ShapeDtypeStruct((16, 128), jnp.float32),
        scratch_shapes=[
            pltpu.VMEM((8, 128), jnp.float32),
            pltpu.VMEM((NP, 128), jnp.float32),
            pltpu.VMEM((16, 128), jnp.float32),
        ],
    )(hp, hj, w4, pv, wl, bcol)


def _knn_and_gather(feat, featT, bcol, brow, rmin, rmax, table, chunk):
    idx = _knn(feat, featT, bcol, brow, rmin, rmax)[:, :K]
    idx = jnp.where(jnp.arange(NP)[:, None] < N, idx, 0)
    idx = jnp.clip(idx, 0, NP - 1)
    idx_flat = idx.T.reshape(-1)
    return _sc_gather(table, idx_flat, chunk)


def kernel(x, batch, W1, b1, g1, be1, W2, b2, g2, be2, W3, b3, g3, be3,
           W4, b4, g4, be4, Wl, bl):
    batch = batch.astype(jnp.int32)
    batchp = jnp.concatenate([batch, jnp.full((NP - N,), G, jnp.int32)])
    bcol = batchp.reshape(NP, 1)
    brow = batchp.reshape(1, NP)
    rmin = batchp[0::T]
    rmax = batchp[T - 1::T]

    xpz = jnp.zeros((NP, 128), jnp.float32).at[:N, :8].set(x)
    xp2 = xpz.at[:N, 8:16].set(x)

    w1 = jnp.zeros((128, 64), jnp.float32).at[:16, :].set(W1)
    pv1 = jnp.zeros((16, 64), jnp.float32)
    pv1 = pv1.at[0].set(b1).at[1].set(g1).at[2].set(be1)
    pv1 = pv1.at[3].set(b2).at[4].set(g2).at[5].set(be2)
    pv1 = pv1.at[6].set(b3).at[7].set(g3).at[8].set(be3)

    xj = _knn_and_gather(xpz, xpz.T, bcol, brow, rmin, rmax, xp2, 800)
    hp = _mlp1(xp2, xj, w1, W2, W3, pv1)

    pv2 = jnp.zeros((8, 128), jnp.float32)
    pv2 = pv2.at[0].set(b4).at[1].set(g4).at[2].set(be4)
    pv2 = pv2.at[3, :2].set(bl)
    wlp = jnp.zeros((128, 128), jnp.float32).at[:, :2].set(Wl)

    h1z = hp.at[:, 64:].set(0.0)
    hj = _knn_and_gather(h1z, h1z.T, bcol, brow, rmin, rmax, hp, 800)
    out = _mlp2(hp, hj, W4, pv2, wlp, bcol)
    return out[:G, :2]
